# baseline (speedup 1.0000x reference)
import jax
import jax.numpy as jnp
from jax import lax
from jax.experimental import pallas as pl
from jax.experimental.pallas import tpu as pltpu
from jax.experimental.pallas import tpu_sc as plsc

NUM_ROWS = 1000000
DIM = 64
BATCH = 4096
HIST = 200

NC = 2
NS = 16
NW = NC * NS
LANES = 16

TOTAL = BATCH * HIST
ROWS_PER_W = TOTAL // NW
CHUNK = 128
NCHUNK = ROWS_PER_W // CHUNK
NBUF = 4

WROWS = TOTAL // 2
TC_BLK = 512


def _sc_body(table_hbm, idx_hbm, out_hbm, idx_v,
             rows_bufs, cast_bufs, gsems, osems):
    wid = lax.axis_index("s") * NC + lax.axis_index("c")
    row0 = wid * ROWS_PER_W
    wrow0 = row0 // 2

    pltpu.sync_copy(idx_hbm.at[pl.ds(row0, ROWS_PER_W)], idx_v)

    def gather_copy(j, b):
        return pltpu.make_async_copy(
            table_hbm.at[idx_v.at[pl.ds(j * CHUNK, CHUNK)]], rows_bufs[b],
            gsems[b])

    def out_copy(j, b):
        return pltpu.make_async_copy(
            cast_bufs[b],
            out_hbm.at[pl.ds(wrow0 + j * (CHUNK // 2), CHUNK // 2)],
            osems[b])

    for b in range(NBUF):
        gather_copy(b, b).start()

    def outer(j0, carry):
        for b in range(NBUF):
            j = j0 * NBUF + b
            gather_copy(j, b).wait()

            @pl.when(j0 > 0)
            def _wait_prev_out():
                out_copy(j - NBUF, b).wait()

            src = rows_bufs[b]
            dst = cast_bufs[b]

            @plsc.parallel_loop(0, CHUNK // 2, unroll=8)
            def cast_pair(r2, src=src, dst=dst):
                r = r2 * 2
                for cc in range(0, DIM, LANES):
                    a = src[r, pl.ds(cc, LANES)]
                    bb = src[r + 1, pl.ds(cc, LANES)]
                    w = plsc.pack(a, bb, format=plsc.PackFormat.INTERLEAVED)
                    dst[r2, pl.ds(cc, LANES)] = plsc.bitcast(w, jnp.int32)

            out_copy(j, b).start()

            @pl.when(j0 < NCHUNK // NBUF - 1)
            def _next_gather():
                gather_copy(j + NBUF, b).start()
        return carry

    lax.fori_loop(0, NCHUNK // NBUF, outer, 0)

    for b in range(NBUF):
        out_copy(NCHUNK - NBUF + b, b).wait()


def _tc_repack(w_ref, out_ref):
    x = w_ref[...]
    y = pltpu.bitcast(x, jnp.bfloat16)
    out_ref[...] = y[:, :DIM]


@jax.jit
def _embed(indices, table):
    run = pl.kernel(
        _sc_body,
        out_type=jax.ShapeDtypeStruct((WROWS, 2 * DIM), jnp.int32),
        mesh=plsc.VectorSubcoreMesh(core_axis_name="c", subcore_axis_name="s"),
        compiler_params=pltpu.CompilerParams(
            needs_layout_passes=False, use_tc_tiling_on_sc=False),
        scratch_types=[
            pltpu.VMEM((ROWS_PER_W,), jnp.int32),
            [pltpu.VMEM((CHUNK, DIM), jnp.float32) for _ in range(NBUF)],
            [pltpu.VMEM((CHUNK // 2, 2 * DIM), jnp.int32) for _ in range(NBUF)],
            [pltpu.SemaphoreType.DMA for _ in range(NBUF)],
            [pltpu.SemaphoreType.DMA for _ in range(NBUF)],
        ],
    )
    words = run(table, indices.reshape(-1))

    out = pl.pallas_call(
        _tc_repack,
        out_shape=jax.ShapeDtypeStruct((TOTAL, DIM), jnp.bfloat16),
        grid=(WROWS // TC_BLK,),
        in_specs=[pl.BlockSpec((TC_BLK, 2 * DIM), lambda i: (i, 0))],
        out_specs=pl.BlockSpec((2 * TC_BLK, DIM), lambda i: (i, 0)),
    )(words)
    return out.reshape(BATCH, HIST, DIM)


def kernel(input, embedding_weight):
    return _embed(input, embedding_weight)


# speedup vs baseline: 1.3441x; 1.3441x over previous
import jax
import jax.numpy as jnp
from jax import lax
from jax.experimental import pallas as pl
from jax.experimental.pallas import tpu as pltpu
from jax.experimental.pallas import tpu_sc as plsc

NUM_ROWS = 1000000
DIM = 64
BATCH = 4096
HIST = 200

NC = 2
NS = 16
NW = NC * NS
LANES = 16

TOTAL = BATCH * HIST
ROWS_PER_W = TOTAL // NW
CHUNK = 128
NCHUNK = ROWS_PER_W // CHUNK
NBUF = 4

WROWS = TOTAL // 2
TC_BLK = 4096


def _sc_body(table_hbm, idx_hbm, out_hbm, idx_v,
             rows_bufs, cast_bufs, gsems, osems):
    wid = lax.axis_index("s") * NC + lax.axis_index("c")
    row0 = wid * ROWS_PER_W
    wrow0 = row0 // 2

    pltpu.sync_copy(idx_hbm.at[pl.ds(row0, ROWS_PER_W)], idx_v)

    def gather_copy(j, b):
        return pltpu.make_async_copy(
            table_hbm.at[idx_v.at[pl.ds(j * CHUNK, CHUNK)]], rows_bufs[b],
            gsems[b])

    def out_copy(j, b):
        return pltpu.make_async_copy(
            cast_bufs[b],
            out_hbm.at[pl.ds(wrow0 + j * (CHUNK // 2), CHUNK // 2)],
            osems[b])

    for b in range(NBUF):
        gather_copy(b, b).start()

    def outer(j0, carry):
        for b in range(NBUF):
            j = j0 * NBUF + b
            gather_copy(j, b).wait()

            @pl.when(j0 > 0)
            def _wait_prev_out():
                out_copy(j - NBUF, b).wait()

            src = rows_bufs[b]
            dst = cast_bufs[b]

            @plsc.parallel_loop(0, CHUNK // 2, unroll=8)
            def cast_pair(r2, src=src, dst=dst):
                r = r2 * 2
                for cc in range(0, DIM, LANES):
                    a = src[r, pl.ds(cc, LANES)]
                    bb = src[r + 1, pl.ds(cc, LANES)]
                    w = plsc.pack(a, bb, format=plsc.PackFormat.INTERLEAVED)
                    dst[r2, pl.ds(cc, LANES)] = plsc.bitcast(w, jnp.int32)

            out_copy(j, b).start()

            @pl.when(j0 < NCHUNK // NBUF - 1)
            def _next_gather():
                gather_copy(j + NBUF, b).start()
        return carry

    lax.fori_loop(0, NCHUNK // NBUF, outer, 0)

    for b in range(NBUF):
        out_copy(NCHUNK - NBUF + b, b).wait()


def _tc_repack(w_ref, out_ref):
    x = w_ref[...]
    y = pltpu.bitcast(x, jnp.bfloat16)
    out_ref[...] = y[:, :DIM]


@jax.jit
def _embed(indices, table):
    run = pl.kernel(
        _sc_body,
        out_type=jax.ShapeDtypeStruct((WROWS, 2 * DIM), jnp.int32),
        mesh=plsc.VectorSubcoreMesh(core_axis_name="c", subcore_axis_name="s"),
        compiler_params=pltpu.CompilerParams(
            needs_layout_passes=False, use_tc_tiling_on_sc=False),
        scratch_types=[
            pltpu.VMEM((ROWS_PER_W,), jnp.int32),
            [pltpu.VMEM((CHUNK, DIM), jnp.float32) for _ in range(NBUF)],
            [pltpu.VMEM((CHUNK // 2, 2 * DIM), jnp.int32) for _ in range(NBUF)],
            [pltpu.SemaphoreType.DMA for _ in range(NBUF)],
            [pltpu.SemaphoreType.DMA for _ in range(NBUF)],
        ],
    )
    words = run(table, indices.reshape(-1))

    out = pl.pallas_call(
        _tc_repack,
        out_shape=jax.ShapeDtypeStruct((TOTAL, DIM), jnp.bfloat16),
        grid=(WROWS // TC_BLK,),
        in_specs=[pl.BlockSpec((TC_BLK, 2 * DIM), lambda i: (i, 0))],
        out_specs=pl.BlockSpec((2 * TC_BLK, DIM), lambda i: (i, 0)),
    )(words)
    return out.reshape(BATCH, HIST, DIM)


def kernel(input, embedding_weight):
    return _embed(input, embedding_weight)
